# Initial kernel scaffold; baseline (speedup 1.0000x reference)
import jax
import jax.numpy as jnp
from jax.experimental import pallas as pl


def kernel(h, edge_index, lin_w, att_src, att_dst, bias):
    raise NotImplementedError("write your pallas kernel here")


# speedup vs baseline: 14.4744x; 14.4744x over previous
import functools

import jax
import jax.numpy as jnp
from jax import lax
from jax.experimental import pallas as pl
from jax.experimental.pallas import tpu as pltpu
from jax.experimental.pallas import tpu_sc as plsc

N = 10000
E = 320000
C = 128

NC = 2
NS = 16
L = 16
NW = NC * NS

B = 128
SB = 8
NB = 80
NSB = NB // SB
EP = NW * NB * B
NPAD = 10112
STRIPE = NPAD // NS


def _pre_body(h_ref, w_ref, asrc_ref, adst_ref, hp_ref, as_ref, ad_ref):
    hp = lax.dot_general(h_ref[...], w_ref[...],
                         (((1,), (1,)), ((), ())),
                         preferred_element_type=jnp.float32)
    hp_ref[...] = hp
    as_ref[...] = jnp.sum(hp * asrc_ref[...], axis=1, keepdims=True)
    ad_ref[...] = jnp.sum(hp * adst_ref[...], axis=1, keepdims=True)


def _post_body(acc_ref, den_ref, hp_ref, as_ref, ad_ref, bias_ref, out_ref):
    num = acc_ref[0:N, :] + acc_ref[NPAD:NPAD + N, :]
    den = jnp.sum(den_ref[0:N, :], axis=1, keepdims=True)
    x = as_ref[...] + ad_ref[...]
    w_self = jnp.exp(1.0 / (1.0 + jnp.exp(-x)))
    out_ref[...] = (num + w_self * hp_ref[...]) / (den + w_self) + bias_ref[...]


def _sc_body(hp_hbm, asp_hbm, adp_hbm, src_hbm, dst_hbm, acc_hbm, den_hbm,
             as_v, ad_v, src_v, dst_v, wrow, den_v, gbuf, accum, gsem):
    cid = lax.axis_index("c")
    sid = lax.axis_index("s")
    wid = cid * NS + sid

    pltpu.sync_copy(asp_hbm, as_v)
    pltpu.sync_copy(adp_hbm, ad_v)

    zeros = jnp.zeros((L,), jnp.float32)

    def _zero_row(r, carry):
        for cc in range(C // L):
            gbuf[r, pl.ds(cc * L, L)] = zeros
        return carry

    lax.fori_loop(0, B, _zero_row, 0)

    def _zero_den(i, carry):
        den_v[pl.ds(i * L, L)] = zeros
        return carry

    lax.fori_loop(0, NPAD // L, _zero_den, 0)

    base = sid * STRIPE
    for k in range(STRIPE // B):
        pltpu.sync_copy(gbuf, accum.at[pl.ds(base + k * B, B)])
    rem = STRIPE - (STRIPE // B) * B
    pltpu.sync_copy(gbuf.at[pl.ds(0, rem)],
                    accum.at[pl.ds(base + (STRIPE // B) * B, rem)])
    plsc.subcore_barrier()

    def _super(sb, carry):
        pltpu.sync_copy(src_hbm.at[wid, pl.ds(sb * SB, SB)], src_v)
        pltpu.sync_copy(dst_hbm.at[wid, pl.ds(sb * SB, SB)], dst_v)

        def _batch(jj, c1):
            pltpu.async_copy(hp_hbm.at[src_v.at[jj]], gbuf, gsem).wait()
            for g in range(B // L):
                s16 = src_v[jj, pl.ds(g * L, L)]
                d16 = dst_v[jj, pl.ds(g * L, L)]
                x = plsc.load_gather(as_v, [s16]) + plsc.load_gather(ad_v, [d16])
                sg = 1.0 / (1.0 + jnp.exp(-x))
                w16 = jnp.exp(sg)
                wrow[pl.ds(g * L, L)] = w16
                plsc.addupdate_scatter(den_v, [d16], w16)

            def _scale_row(r, c2):
                wv = plsc.load_gather(wrow, [jnp.full((L,), r, jnp.int32)])
                for cc in range(C // L):
                    gbuf[r, pl.ds(cc * L, L)] = gbuf[r, pl.ds(cc * L, L)] * wv
                return c2

            lax.fori_loop(0, B, _scale_row, 0)
            pltpu.sync_copy(gbuf, accum.at[dst_v.at[jj]], add=True)
            return c1

        lax.fori_loop(0, SB, _batch, 0)
        return carry

    lax.fori_loop(0, NSB, _super, 0)
    pltpu.sync_copy(den_v, den_hbm.at[wid])
    plsc.subcore_barrier()

    pltpu.sync_copy(accum.at[pl.ds(base, STRIPE)],
                    acc_hbm.at[pl.ds(cid * NPAD + base, STRIPE)])


_sc_edges = functools.partial(
    pl.kernel,
    out_type=[
        jax.ShapeDtypeStruct((NC * NPAD, C), jnp.float32),
        jax.ShapeDtypeStruct((NW, NPAD), jnp.float32),
    ],
    mesh=plsc.VectorSubcoreMesh(core_axis_name="c", subcore_axis_name="s",
                                num_cores=NC, num_subcores=NS),
    scratch_types=[
        pltpu.VMEM((NPAD,), jnp.float32),
        pltpu.VMEM((NPAD,), jnp.float32),
        pltpu.VMEM((SB, B), jnp.int32),
        pltpu.VMEM((SB, B), jnp.int32),
        pltpu.VMEM((B,), jnp.float32),
        pltpu.VMEM((NPAD,), jnp.float32),
        pltpu.VMEM((B, C), jnp.float32),
        pltpu.VMEM_SHARED((NPAD, C), jnp.float32),
        pltpu.SemaphoreType.DMA,
    ],
    compiler_params=pltpu.CompilerParams(needs_layout_passes=False),
)(_sc_body)


@jax.jit
def kernel(h, edge_index, lin_w, att_src, att_dst, bias):
    asrc = att_src.reshape(1, C)
    adst = att_dst.reshape(1, C)

    hp, a_s, a_d = pl.pallas_call(
        _pre_body,
        out_shape=[
            jax.ShapeDtypeStruct((N, C), jnp.float32),
            jax.ShapeDtypeStruct((N, 1), jnp.float32),
            jax.ShapeDtypeStruct((N, 1), jnp.float32),
        ],
    )(h, lin_w, asrc, adst)

    asp = jnp.pad(a_s.reshape(N), (0, NPAD - N))
    adp = jnp.pad(a_d.reshape(N), (0, NPAD - N))
    src = jnp.concatenate(
        [edge_index[0], jnp.zeros((EP - E,), jnp.int32)]).reshape(NW, NB, B)
    dst = jnp.concatenate(
        [edge_index[1], jnp.full((EP - E,), N, jnp.int32)]).reshape(NW, NB, B)

    acc, den = _sc_edges(hp, asp, adp, src, dst)

    out = pl.pallas_call(
        _post_body,
        out_shape=jax.ShapeDtypeStruct((N, C), jnp.float32),
    )(acc, den.T, hp, a_s, a_d, bias.reshape(1, C))
    return out


# speedup vs baseline: 18.2627x; 1.2617x over previous
import functools

import jax
import jax.numpy as jnp
from jax import lax
from jax.experimental import pallas as pl
from jax.experimental.pallas import tpu as pltpu
from jax.experimental.pallas import tpu_sc as plsc

N = 10000
E = 320000
C = 128

NC = 2
NS = 16
L = 16
NW = NC * NS

B = 128
SB = 16
NB = 80
NSB = NB // SB
EP = NW * NB * B
NPAD = 10112
STRIPE = NPAD // NS


def _pre_body(h_ref, w_ref, asrc_ref, adst_ref, hp_ref, as_ref, ad_ref):
    hp = lax.dot_general(h_ref[...], w_ref[...],
                         (((1,), (1,)), ((), ())),
                         preferred_element_type=jnp.float32)
    hp_ref[...] = hp
    as_ref[...] = jnp.sum(hp * asrc_ref[...], axis=1, keepdims=True)
    ad_ref[...] = jnp.sum(hp * adst_ref[...], axis=1, keepdims=True)


def _post_body(acc_ref, den_ref, hp_ref, as_ref, ad_ref, bias_ref, out_ref):
    num = acc_ref[0:N, :] + acc_ref[NPAD:NPAD + N, :]
    den = jnp.sum(den_ref[0:N, :], axis=1, keepdims=True)
    x = as_ref[...] + ad_ref[...]
    w_self = jnp.exp(1.0 / (1.0 + jnp.exp(-x)))
    out_ref[...] = (num + w_self * hp_ref[...]) / (den + w_self) + bias_ref[...]


def _sc_body(hp_hbm, asp_hbm, adp_hbm, src_hbm, dst_hbm, acc_hbm, den_hbm,
             src_v, dst_v, wrow, den_v, gbuf0, gbuf1, asb0, asb1, adb0, adb1,
             accum, gsem0, gsem1):
    cid = lax.axis_index("c")
    sid = lax.axis_index("s")
    wid = cid * NS + sid

    zeros = jnp.zeros((L,), jnp.float32)

    def _zero_row(r, carry):
        for cc in range(C // L):
            gbuf0[r, pl.ds(cc * L, L)] = zeros
        return carry

    lax.fori_loop(0, B, _zero_row, 0)

    def _zero_den(i, carry):
        den_v[pl.ds(i * L, L)] = zeros
        return carry

    lax.fori_loop(0, NPAD // L, _zero_den, 0)

    base = sid * STRIPE
    for k in range(STRIPE // B):
        pltpu.sync_copy(gbuf0, accum.at[pl.ds(base + k * B, B)])
    rem = STRIPE - (STRIPE // B) * B
    pltpu.sync_copy(gbuf0.at[pl.ds(0, rem)],
                    accum.at[pl.ds(base + (STRIPE // B) * B, rem)])
    plsc.subcore_barrier()

    def _fire(jj, gbuf, asb, adb, sem):
        pltpu.async_copy(hp_hbm.at[src_v.at[jj]], gbuf, sem)
        pltpu.async_copy(asp_hbm.at[src_v.at[jj]], asb, sem)
        pltpu.async_copy(adp_hbm.at[dst_v.at[jj]], adb, sem)

    def _drain(gbuf, asb, adb, sem):
        pltpu.make_async_copy(hp_hbm.at[src_v.at[0]], gbuf, sem).wait()
        pltpu.make_async_copy(asp_hbm.at[src_v.at[0]], asb, sem).wait()
        pltpu.make_async_copy(adp_hbm.at[dst_v.at[0]], adb, sem).wait()

    def _process(jj, gbuf, asb, adb):
        for g in range(B // L):
            d16 = dst_v[jj, pl.ds(g * L, L)]
            x = asb[pl.ds(g * L, L)] + adb[pl.ds(g * L, L)]
            sg = 1.0 / (1.0 + jnp.exp(-x))
            w16 = jnp.exp(sg)
            wrow[pl.ds(g * L, L)] = w16
            plsc.addupdate_scatter(den_v, [d16], w16)

        def _scale_row(r, c2):
            wv = plsc.load_gather(wrow, [jnp.full((L,), r, jnp.int32)])
            for cc in range(C // L):
                gbuf[r, pl.ds(cc * L, L)] = gbuf[r, pl.ds(cc * L, L)] * wv
            return c2

        lax.fori_loop(0, B, _scale_row, 0)
        pltpu.sync_copy(gbuf, accum.at[dst_v.at[jj]], add=True)

    def _super(sb, carry):
        pltpu.sync_copy(src_hbm.at[wid, pl.ds(sb * SB, SB)], src_v)
        pltpu.sync_copy(dst_hbm.at[wid, pl.ds(sb * SB, SB)], dst_v)
        _fire(0, gbuf0, asb0, adb0, gsem0)

        def _pair(p, c1):
            j0 = 2 * p
            _fire(j0 + 1, gbuf1, asb1, adb1, gsem1)
            _drain(gbuf0, asb0, adb0, gsem0)
            _process(j0, gbuf0, asb0, adb0)

            @pl.when(p < SB // 2 - 1)
            def _():
                _fire(j0 + 2, gbuf0, asb0, adb0, gsem0)

            _drain(gbuf1, asb1, adb1, gsem1)
            _process(j0 + 1, gbuf1, asb1, adb1)
            return c1

        lax.fori_loop(0, SB // 2, _pair, 0)
        return carry

    lax.fori_loop(0, NSB, _super, 0)
    pltpu.sync_copy(den_v, den_hbm.at[wid])
    plsc.subcore_barrier()

    pltpu.sync_copy(accum.at[pl.ds(base, STRIPE)],
                    acc_hbm.at[pl.ds(cid * NPAD + base, STRIPE)])


_sc_edges = functools.partial(
    pl.kernel,
    out_type=[
        jax.ShapeDtypeStruct((NC * NPAD, C), jnp.float32),
        jax.ShapeDtypeStruct((NW, NPAD), jnp.float32),
    ],
    mesh=plsc.VectorSubcoreMesh(core_axis_name="c", subcore_axis_name="s",
                                num_cores=NC, num_subcores=NS),
    scratch_types=[
        pltpu.VMEM((SB, B), jnp.int32),
        pltpu.VMEM((SB, B), jnp.int32),
        pltpu.VMEM((B,), jnp.float32),
        pltpu.VMEM((NPAD,), jnp.float32),
        pltpu.VMEM((B, C), jnp.float32),
        pltpu.VMEM((B, C), jnp.float32),
        pltpu.VMEM((B,), jnp.float32),
        pltpu.VMEM((B,), jnp.float32),
        pltpu.VMEM((B,), jnp.float32),
        pltpu.VMEM((B,), jnp.float32),
        pltpu.VMEM_SHARED((NPAD, C), jnp.float32),
        pltpu.SemaphoreType.DMA,
        pltpu.SemaphoreType.DMA,
    ],
    compiler_params=pltpu.CompilerParams(needs_layout_passes=False),
)(_sc_body)


@jax.jit
def kernel(h, edge_index, lin_w, att_src, att_dst, bias):
    asrc = att_src.reshape(1, C)
    adst = att_dst.reshape(1, C)

    hp, a_s, a_d = pl.pallas_call(
        _pre_body,
        out_shape=[
            jax.ShapeDtypeStruct((N, C), jnp.float32),
            jax.ShapeDtypeStruct((N, 1), jnp.float32),
            jax.ShapeDtypeStruct((N, 1), jnp.float32),
        ],
    )(h, lin_w, asrc, adst)

    asp = jnp.pad(a_s.reshape(N), (0, NPAD - N))
    adp = jnp.pad(a_d.reshape(N), (0, NPAD - N))
    src = jnp.concatenate(
        [edge_index[0], jnp.zeros((EP - E,), jnp.int32)]).reshape(NW, NB, B)
    dst = jnp.concatenate(
        [edge_index[1], jnp.full((EP - E,), N, jnp.int32)]).reshape(NW, NB, B)

    acc, den = _sc_edges(hp, asp, adp, src, dst)

    out = pl.pallas_call(
        _post_body,
        out_shape=jax.ShapeDtypeStruct((N, C), jnp.float32),
    )(acc, den.T, hp, a_s, a_d, bias.reshape(1, C))
    return out


# speedup vs baseline: 21.0001x; 1.1499x over previous
import functools

import jax
import jax.numpy as jnp
from jax import lax
from jax.experimental import pallas as pl
from jax.experimental.pallas import tpu as pltpu
from jax.experimental.pallas import tpu_sc as plsc

N = 10000
E = 320000
C = 128

NC = 2
NS = 16
L = 16
NW = NC * NS

B = 64
NBUF = 4
SB = 8
NB = 160
NSB = NB // SB
EP = NW * NB * B
NPAD = 10112
STRIPE = NPAD // NS


def _pre_body(h_ref, w_ref, asrc_ref, adst_ref, hp_ref, as_ref, ad_ref):
    hp = lax.dot_general(h_ref[...], w_ref[...],
                         (((1,), (1,)), ((), ())),
                         preferred_element_type=jnp.float32)
    hp_ref[...] = hp
    as_ref[...] = jnp.sum(hp * asrc_ref[...], axis=1, keepdims=True)
    ad_ref[...] = jnp.sum(hp * adst_ref[...], axis=1, keepdims=True)


def _post_body(acc_ref, den_ref, hp_ref, as_ref, ad_ref, bias_ref, out_ref):
    num = acc_ref[0:N, :] + acc_ref[NPAD:NPAD + N, :]
    den = jnp.sum(den_ref[0:N, :], axis=1, keepdims=True)
    x = as_ref[...] + ad_ref[...]
    w_self = jnp.exp(1.0 / (1.0 + jnp.exp(-x)))
    out_ref[...] = (num + w_self * hp_ref[...]) / (den + w_self) + bias_ref[...]


def _sc_body(hp_hbm, asp_hbm, adp_hbm, src_hbm, dst_hbm, acc_hbm, den_hbm,
             src_v, dst_v, wrow, den_v,
             gb0, gb1, gb2, gb3, as0, as1, as2, as3, ad0, ad1, ad2, ad3,
             accum, sem0, sem1, sem2, sem3):
    cid = lax.axis_index("c")
    sid = lax.axis_index("s")
    wid = cid * NS + sid
    gbufs = (gb0, gb1, gb2, gb3)
    asbs = (as0, as1, as2, as3)
    adbs = (ad0, ad1, ad2, ad3)
    sems = (sem0, sem1, sem2, sem3)
    gbuf0 = gb0

    zeros = jnp.zeros((L,), jnp.float32)

    def _zero_row(r, carry):
        for cc in range(C // L):
            gbuf0[r, pl.ds(cc * L, L)] = zeros
        return carry

    lax.fori_loop(0, B, _zero_row, 0)

    def _zero_den(i, carry):
        den_v[pl.ds(i * L, L)] = zeros
        return carry

    lax.fori_loop(0, NPAD // L, _zero_den, 0)

    base = sid * STRIPE
    for k in range(STRIPE // B):
        pltpu.sync_copy(gbuf0, accum.at[pl.ds(base + k * B, B)])
    rem = STRIPE - (STRIPE // B) * B
    pltpu.sync_copy(gbuf0.at[pl.ds(0, rem)],
                    accum.at[pl.ds(base + (STRIPE // B) * B, rem)])
    plsc.subcore_barrier()

    def _fire(jj, gbuf, asb, adb, sem):
        pltpu.async_copy(hp_hbm.at[src_v.at[jj]], gbuf, sem)
        pltpu.async_copy(asp_hbm.at[src_v.at[jj]], asb, sem)
        pltpu.async_copy(adp_hbm.at[dst_v.at[jj]], adb, sem)

    def _drain(gbuf, asb, adb, sem):
        pltpu.make_async_copy(hp_hbm.at[src_v.at[0]], gbuf, sem).wait()
        pltpu.make_async_copy(asp_hbm.at[src_v.at[0]], asb, sem).wait()
        pltpu.make_async_copy(adp_hbm.at[dst_v.at[0]], adb, sem).wait()

    def _process(jj, gbuf, asb, adb):
        for g in range(B // L):
            d16 = dst_v[jj, pl.ds(g * L, L)]
            x = asb[pl.ds(g * L, L)] + adb[pl.ds(g * L, L)]
            sg = 1.0 / (1.0 + jnp.exp(-x))
            w16 = jnp.exp(sg)
            wrow[pl.ds(g * L, L)] = w16
            plsc.addupdate_scatter(den_v, [d16], w16)

        def _scale_row(r, c2):
            wv = plsc.load_gather(wrow, [jnp.full((L,), r, jnp.int32)])
            for cc in range(C // L):
                gbuf[r, pl.ds(cc * L, L)] = gbuf[r, pl.ds(cc * L, L)] * wv
            return c2

        lax.fori_loop(0, B, _scale_row, 0)
        pltpu.sync_copy(gbuf, accum.at[dst_v.at[jj]], add=True)

    def _super(sb, carry):
        pltpu.sync_copy(src_hbm.at[wid, pl.ds(sb * SB, SB)], src_v)
        pltpu.sync_copy(dst_hbm.at[wid, pl.ds(sb * SB, SB)], dst_v)
        for b in range(NBUF):
            _fire(b, gbufs[b], asbs[b], adbs[b], sems[b])

        def _grp(q, c1):
            for b in range(NBUF):
                j = q * NBUF + b
                _drain(gbufs[b], asbs[b], adbs[b], sems[b])
                _process(j, gbufs[b], asbs[b], adbs[b])

                @pl.when(j + NBUF < SB)
                def _():
                    _fire(j + NBUF, gbufs[b], asbs[b], adbs[b], sems[b])
            return c1

        lax.fori_loop(0, SB // NBUF, _grp, 0)
        return carry

    lax.fori_loop(0, NSB, _super, 0)
    pltpu.sync_copy(den_v, den_hbm.at[wid])
    plsc.subcore_barrier()

    pltpu.sync_copy(accum.at[pl.ds(base, STRIPE)],
                    acc_hbm.at[pl.ds(cid * NPAD + base, STRIPE)])


_sc_edges = functools.partial(
    pl.kernel,
    out_type=[
        jax.ShapeDtypeStruct((NC * NPAD, C), jnp.float32),
        jax.ShapeDtypeStruct((NW, NPAD), jnp.float32),
    ],
    mesh=plsc.VectorSubcoreMesh(core_axis_name="c", subcore_axis_name="s",
                                num_cores=NC, num_subcores=NS),
    scratch_types=[
        pltpu.VMEM((SB, B), jnp.int32),
        pltpu.VMEM((SB, B), jnp.int32),
        pltpu.VMEM((B,), jnp.float32),
        pltpu.VMEM((NPAD,), jnp.float32),
        pltpu.VMEM((B, C), jnp.float32),
        pltpu.VMEM((B, C), jnp.float32),
        pltpu.VMEM((B, C), jnp.float32),
        pltpu.VMEM((B, C), jnp.float32),
        pltpu.VMEM((B,), jnp.float32),
        pltpu.VMEM((B,), jnp.float32),
        pltpu.VMEM((B,), jnp.float32),
        pltpu.VMEM((B,), jnp.float32),
        pltpu.VMEM((B,), jnp.float32),
        pltpu.VMEM((B,), jnp.float32),
        pltpu.VMEM((B,), jnp.float32),
        pltpu.VMEM((B,), jnp.float32),
        pltpu.VMEM_SHARED((NPAD, C), jnp.float32),
        pltpu.SemaphoreType.DMA,
        pltpu.SemaphoreType.DMA,
        pltpu.SemaphoreType.DMA,
        pltpu.SemaphoreType.DMA,
    ],
    compiler_params=pltpu.CompilerParams(needs_layout_passes=False),
)(_sc_body)


@jax.jit
def kernel(h, edge_index, lin_w, att_src, att_dst, bias):
    asrc = att_src.reshape(1, C)
    adst = att_dst.reshape(1, C)

    hp, a_s, a_d = pl.pallas_call(
        _pre_body,
        out_shape=[
            jax.ShapeDtypeStruct((N, C), jnp.float32),
            jax.ShapeDtypeStruct((N, 1), jnp.float32),
            jax.ShapeDtypeStruct((N, 1), jnp.float32),
        ],
    )(h, lin_w, asrc, adst)

    asp = jnp.pad(a_s.reshape(N), (0, NPAD - N))
    adp = jnp.pad(a_d.reshape(N), (0, NPAD - N))
    src = jnp.concatenate(
        [edge_index[0], jnp.zeros((EP - E,), jnp.int32)]).reshape(NW, NB, B)
    dst = jnp.concatenate(
        [edge_index[1], jnp.full((EP - E,), N, jnp.int32)]).reshape(NW, NB, B)

    acc, den = _sc_edges(hp, asp, adp, src, dst)

    out = pl.pallas_call(
        _post_body,
        out_shape=jax.ShapeDtypeStruct((N, C), jnp.float32),
    )(acc, den.T, hp, a_s, a_d, bias.reshape(1, C))
    return out
